# baseline (speedup 1.0000x reference)
import functools

import jax
import jax.numpy as jnp
from jax import lax
from jax.experimental import pallas as pl
from jax.experimental.pallas import tpu as pltpu
from jax.experimental.pallas import tpu_sc as plsc

VOCAB_SIZE = 1000000
EMBED_DIM = 64
WPR = EMBED_DIM // 2
BATCH = 16384
CTX_LEN = 21

NC = 2
NS = 16
LANES = 16
NW = NC * NS

SUB = 64
GROUPS_TOTAL = BATCH // SUB
G_PER_W = GROUPS_TOTAL // NW
PAIRS = SUB * CTX_LEN
IDX_CHUNK = 112
N_CHUNKS = PAIRS // IDX_CHUNK

CONV_C = 2048
CONV_R = 4 * CONV_C
CONV_GRID = (VOCAB_SIZE + CONV_R - 1) // CONV_R
VOCAB_PAD = CONV_GRID * CONV_R


def _pack_bf16(x):
    xb = x.astype(jnp.bfloat16).reshape(WPR, 2, CONV_C)
    lo = lax.bitcast_convert_type(xb[:, 0, :], jnp.uint16).astype(jnp.uint32)
    hi = lax.bitcast_convert_type(xb[:, 1, :], jnp.uint16).astype(jnp.uint32)
    return (lo | (hi << 16)).astype(jnp.int32)


def _tc_convert_body(a0, a1, a2, a3, b0, b1, b2, b3, co_ref, xo_ref):
    c4 = jnp.concatenate([_pack_bf16(a0[...]), _pack_bf16(a1[...]),
                          _pack_bf16(a2[...]), _pack_bf16(a3[...])], axis=0)
    co_ref[...] = c4.T.reshape(CONV_C * 128)
    x4 = jnp.concatenate([_pack_bf16(b0[...]), _pack_bf16(b1[...]),
                          _pack_bf16(b2[...]), _pack_bf16(b3[...])], axis=0)
    xo_ref[...] = x4.T.reshape(CONV_C * 128)


def _tc_convert(ct, xt):
    last = (VOCAB_SIZE - 1) // CONV_C
    specs = [
        pl.BlockSpec((EMBED_DIM, CONV_C),
                     functools.partial(
                         lambda h, i: (0, jnp.minimum(4 * i + h, last)), h))
        for h in range(4)
    ]
    out_spec = pl.BlockSpec((CONV_C * 128,), lambda i: (i,))
    return pl.pallas_call(
        _tc_convert_body,
        grid=(CONV_GRID,),
        in_specs=specs + specs,
        out_specs=[out_spec, out_spec],
        out_shape=[
            jax.ShapeDtypeStruct((VOCAB_PAD * WPR,), jnp.int32),
            jax.ShapeDtypeStruct((VOCAB_PAD * WPR,), jnp.int32),
        ],
    )(ct, ct, ct, ct, xt, xt, xt, xt)


def _remap_rows(r):
    q = r % CONV_R
    return (r - q) + 4 * (q % CONV_C) + q // CONV_C


_GATHER_DN = lax.GatherDimensionNumbers(
    offset_dims=(), collapsed_slice_dims=(0,), start_index_map=(0,))


def _lane_sum(v, perms):
    for idx in perms:
        v = v + lax.gather(v, idx, _GATHER_DN, slice_sizes=(1,),
                           mode=lax.GatherScatterMode.PROMISE_IN_BOUNDS)
    return v


_HI_MASK = -65536


def _unpack2(w):
    ev = lax.bitcast_convert_type(lax.shift_left(w, 16), jnp.float32)
    od = lax.bitcast_convert_type(w & _HI_MASK, jnp.float32)
    return ev, od


def _sc_body(cw_hbm, cn_hbm, ctr_tab, ctx_tab, out_hbm,
             cw_idx, cn_idx, ctr_rows, ctx_rows, out_pad, sem):
    wid = lax.axis_index("s") * NC + lax.axis_index("c")
    lane_ids = lax.iota(jnp.int32, LANES)
    perms = [(lane_ids ^ k).reshape(LANES, 1) for k in (8, 4, 2, 1)]

    for g_local in range(G_PER_W):
        g = wid * G_PER_W + g_local
        pltpu.sync_copy(cw_hbm.at[g], cw_idx)
        pltpu.sync_copy(cn_hbm.at[g], cn_idx)

        descs = [pltpu.async_copy(ctr_tab.at[cw_idx], ctr_rows, sem)]
        for i in range(N_CHUNKS):
            descs.append(pltpu.async_copy(
                ctx_tab.at[cn_idx.at[i]],
                ctx_rows.at[pl.ds(i * IDX_CHUNK, IDX_CHUNK)], sem))
        for d in descs:
            d.wait()

        def body(b, carry):
            ce0, co0 = _unpack2(ctr_rows[b, pl.ds(0, LANES)])
            ce1, co1 = _unpack2(ctr_rows[b, pl.ds(LANES, LANES)])
            v0 = jnp.zeros((LANES,), jnp.float32)
            v1 = jnp.zeros((LANES,), jnp.float32)
            for j in range(CTX_LEN):
                r = b * CTX_LEN + j
                te0, to0 = _unpack2(ctx_rows[r, pl.ds(0, LANES)])
                te1, to1 = _unpack2(ctx_rows[r, pl.ds(LANES, LANES)])
                acc = ce0 * te0 + co0 * to0 + ce1 * te1 + co1 * to1
                s = _lane_sum(acc, perms)
                if j < LANES:
                    v0 = jnp.where(lane_ids == j, s, v0)
                else:
                    v1 = jnp.where(lane_ids == (j - LANES), s, v1)
            out_pad[b, pl.ds(0, LANES)] = v0
            out_pad[b, pl.ds(LANES, LANES)] = v1
            return carry

        lax.fori_loop(0, SUB, body, 0)

        pltpu.sync_copy(out_pad, out_hbm.at[pl.ds(g * SUB, SUB)])


@jax.jit
def _run(cw_g, cn_g, ctr_tab, ctx_tab):
    mesh = plsc.VectorSubcoreMesh(
        core_axis_name="c", subcore_axis_name="s",
        num_cores=NC, num_subcores=NS)
    f = pl.kernel(
        _sc_body,
        out_type=jax.ShapeDtypeStruct((BATCH, 2 * LANES), jnp.float32),
        mesh=mesh,
        scratch_types=[
            pltpu.VMEM((SUB,), jnp.int32),
            pltpu.VMEM((N_CHUNKS, IDX_CHUNK), jnp.int32),
            pltpu.VMEM((SUB, WPR), jnp.int32),
            pltpu.VMEM((PAIRS, WPR), jnp.int32),
            pltpu.VMEM((SUB, 2 * LANES), jnp.float32),
            pltpu.SemaphoreType.DMA,
        ],
        compiler_params=pltpu.CompilerParams(
            use_tc_tiling_on_sc=False, needs_layout_passes=False),
    )
    return f(cw_g, cn_g, ctr_tab, ctx_tab)


def kernel(center_words, context_negatives, center_embeddings, context_embeddings):
    cw_g = _remap_rows(center_words.astype(jnp.int32)).reshape(GROUPS_TOTAL, SUB)
    cn_g = _remap_rows(context_negatives.astype(jnp.int32)).reshape(
        GROUPS_TOTAL, N_CHUNKS, IDX_CHUNK)
    ctr_pk, ctx_pk = _tc_convert(center_embeddings.T, context_embeddings.T)
    out = _run(cw_g, cn_g, ctr_pk.reshape(VOCAB_PAD, WPR),
               ctx_pk.reshape(VOCAB_PAD, WPR))
    return out[:, :CTX_LEN].reshape(BATCH, 1, CTX_LEN)


# speedup vs baseline: 1.4394x; 1.4394x over previous
import functools

import jax
import jax.numpy as jnp
from jax import lax
from jax.experimental import pallas as pl
from jax.experimental.pallas import tpu as pltpu
from jax.experimental.pallas import tpu_sc as plsc

VOCAB_SIZE = 1000000
EMBED_DIM = 64
BATCH = 16384
CTX_LEN = 21

NC = 2
NS = 16
LANES = 16
NW = NC * NS

SUB = 64
GROUPS_TOTAL = BATCH // SUB
G_PER_W = GROUPS_TOTAL // NW
PAIRS = SUB * CTX_LEN
IDX_CHUNK = 112
N_CHUNKS = PAIRS // IDX_CHUNK


_GATHER_DN = lax.GatherDimensionNumbers(
    offset_dims=(), collapsed_slice_dims=(0,), start_index_map=(0,))


def _lane_sum(v, perms):
    for idx in perms:
        v = v + lax.gather(v, idx, _GATHER_DN, slice_sizes=(1,),
                           mode=lax.GatherScatterMode.PROMISE_IN_BOUNDS)
    return v


def _sc_body(cw_hbm, cn_hbm, ctr_tab, ctx_tab, out_hbm,
             cw_idx, cn_idx, ctr_rows, ctx_rows, out_pad, sem):
    wid = lax.axis_index("s") * NC + lax.axis_index("c")
    lane_ids = lax.iota(jnp.int32, LANES)
    perms = [(lane_ids ^ k).reshape(LANES, 1) for k in (8, 4, 2, 1)]

    for g_local in range(G_PER_W):
        g = wid * G_PER_W + g_local
        pltpu.sync_copy(cw_hbm.at[g], cw_idx)
        pltpu.sync_copy(cn_hbm.at[g], cn_idx)

        descs = [pltpu.async_copy(ctr_tab.at[cw_idx], ctr_rows, sem)]
        for i in range(N_CHUNKS):
            descs.append(pltpu.async_copy(
                ctx_tab.at[cn_idx.at[i]],
                ctx_rows.at[pl.ds(i * IDX_CHUNK, IDX_CHUNK)], sem))
        for d in descs:
            d.wait()

        def body(b, carry):
            c0 = ctr_rows[b, pl.ds(0, LANES)]
            c1 = ctr_rows[b, pl.ds(16, LANES)]
            c2 = ctr_rows[b, pl.ds(32, LANES)]
            c3 = ctr_rows[b, pl.ds(48, LANES)]
            v0 = jnp.zeros((LANES,), jnp.float32)
            v1 = jnp.zeros((LANES,), jnp.float32)
            for j in range(CTX_LEN):
                r = b * CTX_LEN + j
                acc = (c0 * ctx_rows[r, pl.ds(0, LANES)]
                       + c1 * ctx_rows[r, pl.ds(16, LANES)]
                       + c2 * ctx_rows[r, pl.ds(32, LANES)]
                       + c3 * ctx_rows[r, pl.ds(48, LANES)])
                s = _lane_sum(acc, perms)
                if j < LANES:
                    v0 = jnp.where(lane_ids == j, s, v0)
                else:
                    v1 = jnp.where(lane_ids == (j - LANES), s, v1)
            out_pad[b, pl.ds(0, LANES)] = v0
            out_pad[b, pl.ds(LANES, LANES)] = v1
            return carry

        lax.fori_loop(0, SUB, body, 0)

        pltpu.sync_copy(out_pad, out_hbm.at[pl.ds(g * SUB, SUB)])


CONV_C = 2048
CONV_R = 2 * CONV_C
CONV_GRID = (VOCAB_SIZE + CONV_R - 1) // CONV_R
VOCAB_PAD = CONV_GRID * CONV_R


def _tc_convert_body(ca_ref, cb_ref, xa_ref, xb_ref, co_ref, xo_ref):
    c2 = jnp.concatenate([ca_ref[...], cb_ref[...]], axis=0).T
    co_ref[...] = c2.reshape(CONV_C * 2 * EMBED_DIM)
    x2 = jnp.concatenate([xa_ref[...], xb_ref[...]], axis=0).T
    xo_ref[...] = x2.reshape(CONV_C * 2 * EMBED_DIM)


def _tc_convert(ct, xt):
    last = (VOCAB_SIZE - 1) // CONV_C
    spec_a = pl.BlockSpec((EMBED_DIM, CONV_C),
                          lambda i: (0, jnp.minimum(2 * i, last)))
    spec_b = pl.BlockSpec((EMBED_DIM, CONV_C),
                          lambda i: (0, jnp.minimum(2 * i + 1, last)))
    out_spec = pl.BlockSpec((CONV_R * EMBED_DIM,), lambda i: (i,))
    return pl.pallas_call(
        _tc_convert_body,
        grid=(CONV_GRID,),
        in_specs=[spec_a, spec_b, spec_a, spec_b],
        out_specs=[out_spec, out_spec],
        out_shape=[
            jax.ShapeDtypeStruct((VOCAB_PAD * EMBED_DIM,), jnp.float32),
            jax.ShapeDtypeStruct((VOCAB_PAD * EMBED_DIM,), jnp.float32),
        ],
    )(ct, ct, xt, xt)


def _remap_rows(r):
    q = r % CONV_R
    return (r - q) + 2 * (q % CONV_C) + q // CONV_C


@jax.jit
def _run(cw_g, cn_g, center_embeddings, context_embeddings):
    mesh = plsc.VectorSubcoreMesh(
        core_axis_name="c", subcore_axis_name="s",
        num_cores=NC, num_subcores=NS)
    f = pl.kernel(
        _sc_body,
        out_type=jax.ShapeDtypeStruct((BATCH, 2 * LANES), jnp.float32),
        mesh=mesh,
        scratch_types=[
            pltpu.VMEM((SUB,), jnp.int32),
            pltpu.VMEM((N_CHUNKS, IDX_CHUNK), jnp.int32),
            pltpu.VMEM((SUB, EMBED_DIM), jnp.float32),
            pltpu.VMEM((PAIRS, EMBED_DIM), jnp.float32),
            pltpu.VMEM((SUB, 2 * LANES), jnp.float32),
            pltpu.SemaphoreType.DMA,
        ],
        compiler_params=pltpu.CompilerParams(
            use_tc_tiling_on_sc=False, needs_layout_passes=False),
    )
    return f(cw_g, cn_g, center_embeddings, context_embeddings)


def kernel(center_words, context_negatives, center_embeddings, context_embeddings):
    cw_g = _remap_rows(center_words.astype(jnp.int32)).reshape(GROUPS_TOTAL, SUB)
    cn_g = _remap_rows(context_negatives.astype(jnp.int32)).reshape(
        GROUPS_TOTAL, N_CHUNKS, IDX_CHUNK)
    ctr_bf, ctx_bf = _tc_convert(center_embeddings.T, context_embeddings.T)
    out = _run(cw_g, cn_g, ctr_bf.reshape(VOCAB_PAD, EMBED_DIM),
               ctx_bf.reshape(VOCAB_PAD, EMBED_DIM))
    return out[:, :CTX_LEN].reshape(BATCH, 1, CTX_LEN)


# speedup vs baseline: 1.6256x; 1.1293x over previous
import functools

import jax
import jax.numpy as jnp
from jax import lax
from jax.experimental import pallas as pl
from jax.experimental.pallas import tpu as pltpu
from jax.experimental.pallas import tpu_sc as plsc

VOCAB_SIZE = 1000000
EMBED_DIM = 64
BATCH = 16384
CTX_LEN = 21

NC = 2
NS = 16
LANES = 16
NW = NC * NS

SUB = 64
GROUPS_TOTAL = BATCH // SUB
G_PER_W = GROUPS_TOTAL // NW
PAIRS = SUB * CTX_LEN
IDX_CHUNK = 112
N_CHUNKS = PAIRS // IDX_CHUNK


_GATHER_DN = lax.GatherDimensionNumbers(
    offset_dims=(), collapsed_slice_dims=(0,), start_index_map=(0,))


def _lane_sum(v, perms):
    for idx in perms:
        v = v + lax.gather(v, idx, _GATHER_DN, slice_sizes=(1,),
                           mode=lax.GatherScatterMode.PROMISE_IN_BOUNDS)
    return v


def _sc_body(cw_hbm, cn_hbm, ctr_tab, ctx_tab, out_hbm,
             cw_idx, cn_idx, ctr_rows, ctx_rows, out_pad, sem):
    wid = lax.axis_index("s") * NC + lax.axis_index("c")
    lane_ids = lax.iota(jnp.int32, LANES)
    perms = [(lane_ids ^ k).reshape(LANES, 1) for k in (8, 4, 2, 1)]

    for g_local in range(G_PER_W):
        g = wid * G_PER_W + g_local
        pltpu.sync_copy(cw_hbm.at[g], cw_idx)
        pltpu.sync_copy(cn_hbm.at[g], cn_idx)

        descs = [pltpu.async_copy(ctr_tab.at[cw_idx], ctr_rows, sem)]
        for i in range(N_CHUNKS):
            descs.append(pltpu.async_copy(
                ctx_tab.at[cn_idx.at[i]],
                ctx_rows.at[pl.ds(i * IDX_CHUNK, IDX_CHUNK)], sem))
        for d in descs:
            d.wait()

        def body(b, carry):
            c0 = ctr_rows[b, pl.ds(0, LANES)]
            c1 = ctr_rows[b, pl.ds(16, LANES)]
            c2 = ctr_rows[b, pl.ds(32, LANES)]
            c3 = ctr_rows[b, pl.ds(48, LANES)]
            v0 = jnp.zeros((LANES,), jnp.float32)
            v1 = jnp.zeros((LANES,), jnp.float32)
            for j in range(CTX_LEN):
                r = b * CTX_LEN + j
                acc = (c0 * ctx_rows[r, pl.ds(0, LANES)]
                       + c1 * ctx_rows[r, pl.ds(16, LANES)]
                       + c2 * ctx_rows[r, pl.ds(32, LANES)]
                       + c3 * ctx_rows[r, pl.ds(48, LANES)])
                s = _lane_sum(acc, perms)
                if j < LANES:
                    v0 = jnp.where(lane_ids == j, s, v0)
                else:
                    v1 = jnp.where(lane_ids == (j - LANES), s, v1)
            out_pad[b, pl.ds(0, LANES)] = v0
            out_pad[b, pl.ds(LANES, LANES)] = v1
            return carry

        lax.fori_loop(0, SUB, body, 0)

        pltpu.sync_copy(out_pad, out_hbm.at[pl.ds(g * SUB, SUB)])


CONV_C = 4096
CONV_R = 2 * CONV_C
CONV_GRID = (VOCAB_SIZE + CONV_R - 1) // CONV_R
VOCAB_PAD = CONV_GRID * CONV_R


def _tc_convert_body(ca_ref, cb_ref, xa_ref, xb_ref, co_ref, xo_ref):
    c2 = jnp.concatenate([ca_ref[...], cb_ref[...]], axis=0).T
    co_ref[...] = c2.reshape(CONV_C * 2 * EMBED_DIM)
    x2 = jnp.concatenate([xa_ref[...], xb_ref[...]], axis=0).T
    xo_ref[...] = x2.reshape(CONV_C * 2 * EMBED_DIM)


def _tc_convert(ct, xt):
    last = (VOCAB_SIZE - 1) // CONV_C
    spec_a = pl.BlockSpec((EMBED_DIM, CONV_C),
                          lambda i: (0, jnp.minimum(2 * i, last)))
    spec_b = pl.BlockSpec((EMBED_DIM, CONV_C),
                          lambda i: (0, jnp.minimum(2 * i + 1, last)))
    out_spec = pl.BlockSpec((CONV_R * EMBED_DIM,), lambda i: (i,))
    return pl.pallas_call(
        _tc_convert_body,
        grid=(CONV_GRID,),
        in_specs=[spec_a, spec_b, spec_a, spec_b],
        out_specs=[out_spec, out_spec],
        out_shape=[
            jax.ShapeDtypeStruct((VOCAB_PAD * EMBED_DIM,), jnp.float32),
            jax.ShapeDtypeStruct((VOCAB_PAD * EMBED_DIM,), jnp.float32),
        ],
    )(ct, ct, xt, xt)


def _remap_rows(r):
    q = r % CONV_R
    return (r - q) + 2 * (q % CONV_C) + q // CONV_C


@jax.jit
def _run(cw_g, cn_g, center_embeddings, context_embeddings):
    mesh = plsc.VectorSubcoreMesh(
        core_axis_name="c", subcore_axis_name="s",
        num_cores=NC, num_subcores=NS)
    f = pl.kernel(
        _sc_body,
        out_type=jax.ShapeDtypeStruct((BATCH, 2 * LANES), jnp.float32),
        mesh=mesh,
        scratch_types=[
            pltpu.VMEM((SUB,), jnp.int32),
            pltpu.VMEM((N_CHUNKS, IDX_CHUNK), jnp.int32),
            pltpu.VMEM((SUB, EMBED_DIM), jnp.float32),
            pltpu.VMEM((PAIRS, EMBED_DIM), jnp.float32),
            pltpu.VMEM((SUB, 2 * LANES), jnp.float32),
            pltpu.SemaphoreType.DMA,
        ],
        compiler_params=pltpu.CompilerParams(
            use_tc_tiling_on_sc=False, needs_layout_passes=False),
    )
    return f(cw_g, cn_g, center_embeddings, context_embeddings)


def kernel(center_words, context_negatives, center_embeddings, context_embeddings):
    cw_g = _remap_rows(center_words.astype(jnp.int32)).reshape(GROUPS_TOTAL, SUB)
    cn_g = _remap_rows(context_negatives.astype(jnp.int32)).reshape(
        GROUPS_TOTAL, N_CHUNKS, IDX_CHUNK)
    ctr_bf, ctx_bf = _tc_convert(center_embeddings.T, context_embeddings.T)
    out = _run(cw_g, cn_g, ctr_bf.reshape(VOCAB_PAD, EMBED_DIM),
               ctx_bf.reshape(VOCAB_PAD, EMBED_DIM))
    return out[:, :CTX_LEN].reshape(BATCH, 1, CTX_LEN)


# speedup vs baseline: 1.6480x; 1.0138x over previous
import functools

import jax
import jax.numpy as jnp
from jax import lax
from jax.experimental import pallas as pl
from jax.experimental.pallas import tpu as pltpu
from jax.experimental.pallas import tpu_sc as plsc

VOCAB_SIZE = 1000000
EMBED_DIM = 64
BATCH = 16384
CTX_LEN = 21

NC = 2
NS = 16
LANES = 16
NW = NC * NS

SUB = 64
GROUPS_TOTAL = BATCH // SUB
G_PER_W = GROUPS_TOTAL // NW
PAIRS = SUB * CTX_LEN
IDX_CHUNK = 112
N_CHUNKS = PAIRS // IDX_CHUNK


_GATHER_DN = lax.GatherDimensionNumbers(
    offset_dims=(), collapsed_slice_dims=(0,), start_index_map=(0,))


def _lane_sum(v, perms):
    for idx in perms:
        v = v + lax.gather(v, idx, _GATHER_DN, slice_sizes=(1,),
                           mode=lax.GatherScatterMode.PROMISE_IN_BOUNDS)
    return v


def _sc_body(cw_hbm, cn_hbm, ctr_tab, ctx_tab, out_hbm,
             cw_idx, cn_idx, ctr_rows, ctx_rows, out_pad, sem):
    wid = lax.axis_index("s") * NC + lax.axis_index("c")
    lane_ids = lax.iota(jnp.int32, LANES)
    perms = [(lane_ids ^ k).reshape(LANES, 1) for k in (8, 4, 2, 1)]

    for g_local in range(G_PER_W):
        g = wid * G_PER_W + g_local
        pltpu.sync_copy(cw_hbm.at[g], cw_idx)
        pltpu.sync_copy(cn_hbm.at[g], cn_idx)

        descs = [pltpu.async_copy(ctr_tab.at[cw_idx], ctr_rows, sem)]
        for i in range(N_CHUNKS):
            descs.append(pltpu.async_copy(
                ctx_tab.at[cn_idx.at[i]],
                ctx_rows.at[pl.ds(i * IDX_CHUNK, IDX_CHUNK)], sem))
        for d in descs:
            d.wait()

        def body(b, carry):
            c0 = ctr_rows[b, pl.ds(0, LANES)]
            c1 = ctr_rows[b, pl.ds(16, LANES)]
            c2 = ctr_rows[b, pl.ds(32, LANES)]
            c3 = ctr_rows[b, pl.ds(48, LANES)]
            v0 = jnp.zeros((LANES,), jnp.float32)
            v1 = jnp.zeros((LANES,), jnp.float32)
            for j in range(CTX_LEN):
                r = b * CTX_LEN + j
                acc = (c0 * ctx_rows[r, pl.ds(0, LANES)]
                       + c1 * ctx_rows[r, pl.ds(16, LANES)]
                       + c2 * ctx_rows[r, pl.ds(32, LANES)]
                       + c3 * ctx_rows[r, pl.ds(48, LANES)])
                s = _lane_sum(acc, perms)
                if j < LANES:
                    v0 = jnp.where(lane_ids == j, s, v0)
                else:
                    v1 = jnp.where(lane_ids == (j - LANES), s, v1)
            out_pad[b, pl.ds(0, LANES)] = v0
            out_pad[b, pl.ds(LANES, LANES)] = v1
            return carry

        lax.fori_loop(0, SUB, body, 0)

        pltpu.sync_copy(out_pad, out_hbm.at[pl.ds(g * SUB, SUB)])


CONV_C = 8192
CONV_R = 2 * CONV_C
CONV_GRID = (VOCAB_SIZE + CONV_R - 1) // CONV_R
VOCAB_PAD = CONV_GRID * CONV_R


def _tc_convert_body(ca_ref, cb_ref, xa_ref, xb_ref, co_ref, xo_ref):
    c2 = jnp.concatenate([ca_ref[...], cb_ref[...]], axis=0).T
    co_ref[...] = c2.reshape(CONV_C * 2 * EMBED_DIM)
    x2 = jnp.concatenate([xa_ref[...], xb_ref[...]], axis=0).T
    xo_ref[...] = x2.reshape(CONV_C * 2 * EMBED_DIM)


def _tc_convert(ct, xt):
    last = (VOCAB_SIZE - 1) // CONV_C
    spec_a = pl.BlockSpec((EMBED_DIM, CONV_C),
                          lambda i: (0, jnp.minimum(2 * i, last)))
    spec_b = pl.BlockSpec((EMBED_DIM, CONV_C),
                          lambda i: (0, jnp.minimum(2 * i + 1, last)))
    out_spec = pl.BlockSpec((CONV_R * EMBED_DIM,), lambda i: (i,))
    return pl.pallas_call(
        _tc_convert_body,
        grid=(CONV_GRID,),
        in_specs=[spec_a, spec_b, spec_a, spec_b],
        out_specs=[out_spec, out_spec],
        out_shape=[
            jax.ShapeDtypeStruct((VOCAB_PAD * EMBED_DIM,), jnp.float32),
            jax.ShapeDtypeStruct((VOCAB_PAD * EMBED_DIM,), jnp.float32),
        ],
    )(ct, ct, xt, xt)


def _remap_rows(r):
    q = r % CONV_R
    return (r - q) + 2 * (q % CONV_C) + q // CONV_C


@jax.jit
def _run(cw_g, cn_g, center_embeddings, context_embeddings):
    mesh = plsc.VectorSubcoreMesh(
        core_axis_name="c", subcore_axis_name="s",
        num_cores=NC, num_subcores=NS)
    f = pl.kernel(
        _sc_body,
        out_type=jax.ShapeDtypeStruct((BATCH, 2 * LANES), jnp.float32),
        mesh=mesh,
        scratch_types=[
            pltpu.VMEM((SUB,), jnp.int32),
            pltpu.VMEM((N_CHUNKS, IDX_CHUNK), jnp.int32),
            pltpu.VMEM((SUB, EMBED_DIM), jnp.float32),
            pltpu.VMEM((PAIRS, EMBED_DIM), jnp.float32),
            pltpu.VMEM((SUB, 2 * LANES), jnp.float32),
            pltpu.SemaphoreType.DMA,
        ],
        compiler_params=pltpu.CompilerParams(
            use_tc_tiling_on_sc=False, needs_layout_passes=False),
    )
    return f(cw_g, cn_g, center_embeddings, context_embeddings)


def kernel(center_words, context_negatives, center_embeddings, context_embeddings):
    cw_g = _remap_rows(center_words.astype(jnp.int32)).reshape(GROUPS_TOTAL, SUB)
    cn_g = _remap_rows(context_negatives.astype(jnp.int32)).reshape(
        GROUPS_TOTAL, N_CHUNKS, IDX_CHUNK)
    ctr_bf, ctx_bf = _tc_convert(center_embeddings.T, context_embeddings.T)
    out = _run(cw_g, cn_g, ctr_bf.reshape(VOCAB_PAD, EMBED_DIM),
               ctx_bf.reshape(VOCAB_PAD, EMBED_DIM))
    return out[:, :CTX_LEN].reshape(BATCH, 1, CTX_LEN)
